# baseline (speedup 1.0000x reference)
import functools

import jax
import jax.numpy as jnp
from jax.experimental import pallas as pl
from jax.experimental.pallas import tpu as pltpu


def _upsample_nearest(x, k):
    x = jnp.repeat(x, k, axis=2)
    x = jnp.repeat(x, k, axis=1)
    return x


def _conv3x3_same(x, w_taps):
    B, Hl, Wl, C = x.shape
    bf16, f32 = jnp.bfloat16, jnp.float32
    xb = x.astype(bf16)
    zr = jnp.zeros((B, 1, Wl, C), bf16)
    xp = jnp.concatenate([zr, xb, zr], axis=1)
    zc = jnp.zeros((B, Hl + 2, 1, C), bf16)
    xp = jnp.concatenate([zc, xp, zc], axis=2)
    acc = jnp.zeros((B * Hl * Wl, C), f32)
    for t, (dy, dx) in enumerate((dy, dx) for dy in range(3) for dx in range(3)):
        win = xp[:, dy:dy + Hl, dx:dx + Wl, :].reshape(B * Hl * Wl, C)
        acc = acc + jnp.dot(win, w_taps[t], preferred_element_type=f32)
    return acc.reshape(B, Hl, Wl, C)


def _avgpool2_mm(x_flat, A):
    B = x_flat.shape[0]
    xb = x_flat.astype(jnp.bfloat16)
    return jnp.stack([
        jnp.dot(A, xb[b], preferred_element_type=jnp.float32)
        for b in range(B)])


def _t_mm(x, ident, dtype=jnp.bfloat16):
    P, C = x.shape
    xb = x.astype(dtype)
    idc = ident.astype(dtype)
    cols = []
    for i in range(0, P, 128):
        c = min(128, P - i)
        cols.append(jax.lax.dot_general(
            xb[i:i + c], idc[:c, :c],
            dimension_numbers=(((0,), (0,)), ((), ())),
            preferred_element_type=jnp.float32))
    return cols[0] if len(cols) == 1 else jnp.concatenate(cols, axis=1)


def _t_mm_batch(x_flat, ident, dtype=None):
    del ident, dtype
    return jnp.transpose(x_flat, (0, 2, 1))


def _msfr_kernel(f_ref, w_ref, a3_ref, a4_ref, u8_ref, id_ref,
                 p2_ref, p3_ref, p4_ref, p5_ref, p6_ref, p5s_ref,
                 *, B, H, W, C):
    f32 = jnp.float32
    P1 = H * W
    F = f_ref[...]
    w = w_ref[...]
    ident = id_ref[...]
    u8 = u8_ref[...]

    for b in range(B):
        ft = jnp.transpose(F[b], (1, 0))
        p2_ref[b] = jnp.dot(ft.astype(jnp.bfloat16), u8,
                            preferred_element_type=f32)

    Fv = F.reshape(B, H, W, C)
    u4 = _upsample_nearest(Fv, 4)
    p3 = _conv3x3_same(u4, w) + u4
    p3f = p3.reshape(B, 16 * P1, C)
    for b in range(B):
        p3_ref[b] = jnp.transpose(p3f[b], (1, 0))

    z4 = _avgpool2_mm(p3f, a3_ref[...])
    p4 = _conv3x3_same(z4.reshape(B, 2 * H, 2 * W, C), w) \
        + _upsample_nearest(Fv, 2)
    p4f = p4.reshape(B, 4 * P1, C)
    for b in range(B):
        p4_ref[b] = jnp.transpose(p4f[b], (1, 0))

    z5 = _avgpool2_mm(p4f, a4_ref[...])
    p5 = _conv3x3_same(z5.reshape(B, H, W, C), w) + Fv
    p5f = p5.reshape(B, P1, C)
    p5s_ref[...] = p5f
    for b in range(B):
        p5_ref[b] = jnp.transpose(p5f[b], (1, 0))

    Wp = W // 2
    rows = []
    for qy in range(H // 2):
        b0, b1 = (2 * qy) * W, (2 * qy + 1) * W
        rows.append(jnp.maximum(
            jnp.maximum(p5s_ref[:, pl.ds(b0, Wp, 2), :],
                        p5s_ref[:, pl.ds(b0 + 1, Wp, 2), :]),
            jnp.maximum(p5s_ref[:, pl.ds(b1, Wp, 2), :],
                        p5s_ref[:, pl.ds(b1 + 1, Wp, 2), :])))
    p6f = jnp.concatenate(rows, axis=1)
    for b in range(B):
        p6_ref[b] = jnp.transpose(p6f[b], (1, 0))


def _pool_matrix(Hl, Wl):
    Pi, Po = Hl * Wl, (Hl // 2) * (Wl // 2)
    qi = jnp.arange(Po)[:, None]
    pi = jnp.arange(Pi)[None, :]
    y, x = pi // Wl, pi % Wl
    qy, qx = qi // (Wl // 2), qi % (Wl // 2)
    hit = (y // 2 == qy) & (x // 2 == qx)
    return jnp.where(hit, 0.25, 0.0).astype(jnp.bfloat16)


def _up8_matrix(H, W):
    Pi, Po = H * W, 64 * H * W
    pi = jnp.arange(Pi)[:, None]
    qi = jnp.arange(Po)[None, :]
    Y, X = qi // (8 * W), qi % (8 * W)
    hit = pi == (Y // 8) * W + (X // 8)
    return jnp.where(hit, 1.0, 0.0).astype(jnp.bfloat16)


def _msfr_pallas(f_flat, consts, N, C, H, W, B, interpret=False):
    P1 = H * W

    def blk_in(P):
        return pl.BlockSpec((B, P, C), lambda n: (n, 0, 0))

    def blk_out(P):
        return pl.BlockSpec((B, C, P), lambda n: (n, 0, 0))

    out_shape = (
        jax.ShapeDtypeStruct((N, C, 64 * P1), jnp.float32),
        jax.ShapeDtypeStruct((N, C, 16 * P1), jnp.float32),
        jax.ShapeDtypeStruct((N, C, 4 * P1), jnp.float32),
        jax.ShapeDtypeStruct((N, C, P1), jnp.float32),
        jax.ShapeDtypeStruct((N, C, P1 // 4), jnp.float32),
    )
    return pl.pallas_call(
        functools.partial(_msfr_kernel, B=B, H=H, W=W, C=C),
        grid=(N // B,),
        in_specs=[blk_in(P1)] + [
            pl.BlockSpec(c.shape, lambda n, _nd=c.ndim: (0,) * _nd)
            for c in consts],
        out_specs=(blk_out(64 * P1), blk_out(16 * P1), blk_out(4 * P1),
                   blk_out(P1), blk_out(P1 // 4)),
        out_shape=out_shape,
        scratch_shapes=[pltpu.VMEM((B, P1, C), jnp.float32)],
        compiler_params=pltpu.CompilerParams(
            dimension_semantics=("parallel",)),
        interpret=interpret,
    )(f_flat, *consts)


def kernel(features_nchw, w_oihw, interpret=False):
    N, C, H, W = features_nchw.shape
    P1 = H * W
    B = 2

    f_flat = (jnp.transpose(features_nchw, (0, 2, 3, 1))
              .reshape(N, P1, C).astype(jnp.float32))
    w_taps = (jnp.transpose(w_oihw, (2, 3, 1, 0))
              .reshape(9, C, C).astype(jnp.bfloat16))
    consts = (w_taps, _pool_matrix(4 * H, 4 * W), _pool_matrix(2 * H, 2 * W),
              _up8_matrix(H, W), jnp.eye(C, dtype=jnp.bfloat16))

    p2f, p3f, p4f, p5f, p6f = _msfr_pallas(f_flat, consts, N, C, H, W, B,
                                           interpret=interpret)

    return [p2f.reshape(N, C, 8 * H, 8 * W), p3f.reshape(N, C, 4 * H, 4 * W),
            p4f.reshape(N, C, 2 * H, 2 * W), p5f.reshape(N, C, H, W),
            p6f.reshape(N, C, H // 2, W // 2)]


# speedup vs baseline: 2.6409x; 2.6409x over previous
import functools

import jax
import jax.numpy as jnp
from jax.experimental import pallas as pl
from jax.experimental.pallas import tpu as pltpu


def _upsample_nearest(x, k):
    x = jnp.repeat(x, k, axis=2)
    x = jnp.repeat(x, k, axis=1)
    return x


def _conv3x3_same(x, w_taps):
    B, Hl, Wl, C = x.shape
    bf16, f32 = jnp.bfloat16, jnp.float32
    xb = x.astype(bf16)
    zr = jnp.zeros((B, 1, Wl, C), bf16)
    xp = jnp.concatenate([zr, xb, zr], axis=1)
    zc = jnp.zeros((B, Hl + 2, 1, C), bf16)
    xp = jnp.concatenate([zc, xp, zc], axis=2)
    acc = jnp.zeros((B * Hl * Wl, C), f32)
    for t, (dy, dx) in enumerate((dy, dx) for dy in range(3) for dx in range(3)):
        win = xp[:, dy:dy + Hl, dx:dx + Wl, :].reshape(B * Hl * Wl, C)
        acc = acc + jnp.dot(win, w_taps[t], preferred_element_type=f32)
    return acc.reshape(B, Hl, Wl, C)


def _avgpool2_mm(x_flat, A):
    B = x_flat.shape[0]
    xb = x_flat.astype(jnp.bfloat16)
    return jnp.stack([
        jnp.dot(A, xb[b], preferred_element_type=jnp.float32)
        for b in range(B)])


def _msfr_kernel(f_ref, w_ref, a3_ref, a4_ref,
                 p2_ref, p3_ref, p4_ref, p5_ref, p6_ref, *, B, H, W, C):
    F = f_ref[...].reshape(B, H, W, C)
    w = w_ref[...]

    p2_ref[...] = _upsample_nearest(F, 8).reshape(B, 64 * H * W, C)

    u4 = _upsample_nearest(F, 4)
    p3 = _conv3x3_same(u4, w) + u4
    p3_ref[...] = p3.reshape(B, 16 * H * W, C)

    z4 = _avgpool2_mm(p3.reshape(B, 16 * H * W, C), a3_ref[...])
    p4 = _conv3x3_same(z4.reshape(B, 2 * H, 2 * W, C), w) \
        + _upsample_nearest(F, 2)
    p4_ref[...] = p4.reshape(B, 4 * H * W, C)

    z5 = _avgpool2_mm(p4.reshape(B, 4 * H * W, C), a4_ref[...])
    p5 = _conv3x3_same(z5.reshape(B, H, W, C), w) + F
    p5_ref[...] = p5.reshape(B, H * W, C)

    Wp = W // 2
    for qy in range(H // 2):
        b0, b1 = (2 * qy) * W, (2 * qy + 1) * W
        m = jnp.maximum(
            jnp.maximum(p5_ref[:, pl.ds(b0, Wp, 2), :],
                        p5_ref[:, pl.ds(b0 + 1, Wp, 2), :]),
            jnp.maximum(p5_ref[:, pl.ds(b1, Wp, 2), :],
                        p5_ref[:, pl.ds(b1 + 1, Wp, 2), :]))
        p6_ref[:, pl.ds(qy * Wp, Wp), :] = m


def _pool_matrix(Hl, Wl):
    Pi, Po = Hl * Wl, (Hl // 2) * (Wl // 2)
    qi = jnp.arange(Po)[:, None]
    pi = jnp.arange(Pi)[None, :]
    y, x = pi // Wl, pi % Wl
    qy, qx = qi // (Wl // 2), qi % (Wl // 2)
    hit = (y // 2 == qy) & (x // 2 == qx)
    return jnp.where(hit, 0.25, 0.0).astype(jnp.bfloat16)


def _msfr_pallas(f_flat, w_taps, a3, a4, N, C, H, W, B):
    P1 = H * W

    def blk(P):
        return pl.BlockSpec((B, P, C), lambda n: (n, 0, 0))

    out_shape = (
        jax.ShapeDtypeStruct((N, 64 * P1, C), jnp.float32),
        jax.ShapeDtypeStruct((N, 16 * P1, C), jnp.float32),
        jax.ShapeDtypeStruct((N, 4 * P1, C), jnp.float32),
        jax.ShapeDtypeStruct((N, P1, C), jnp.float32),
        jax.ShapeDtypeStruct((N, P1 // 4, C), jnp.float32),
    )
    return pl.pallas_call(
        functools.partial(_msfr_kernel, B=B, H=H, W=W, C=C),
        grid=(N // B,),
        in_specs=[
            blk(P1),
            pl.BlockSpec((9, C, C), lambda n: (0, 0, 0)),
            pl.BlockSpec(a3.shape, lambda n: (0, 0)),
            pl.BlockSpec(a4.shape, lambda n: (0, 0)),
        ],
        out_specs=(blk(64 * P1), blk(16 * P1), blk(4 * P1), blk(P1),
                   blk(P1 // 4)),
        out_shape=out_shape,
        compiler_params=pltpu.CompilerParams(
            dimension_semantics=("parallel",)),
    )(f_flat, w_taps, a3, a4)


def kernel(features_nchw, w_oihw):
    N, C, H, W = features_nchw.shape
    P1 = H * W
    B = 4

    f_flat = (jnp.transpose(features_nchw, (0, 2, 3, 1))
              .reshape(N, P1, C).astype(jnp.float32))
    w_taps = (jnp.transpose(w_oihw, (2, 3, 1, 0))
              .reshape(9, C, C).astype(jnp.bfloat16))
    a3 = _pool_matrix(4 * H, 4 * W)
    a4 = _pool_matrix(2 * H, 2 * W)

    p2f, p3f, p4f, p5f, p6f = _msfr_pallas(f_flat, w_taps, a3, a4,
                                           N, C, H, W, B)

    def to_nchw(x_flat, h, w):
        return jnp.transpose(x_flat.reshape(N, h, w, C), (0, 3, 1, 2))

    return [to_nchw(p2f, 8 * H, 8 * W), to_nchw(p3f, 4 * H, 4 * W),
            to_nchw(p4f, 2 * H, 2 * W), to_nchw(p5f, H, W),
            to_nchw(p6f, H // 2, W // 2)]


# speedup vs baseline: 2.7025x; 1.0233x over previous
import functools

import jax
import jax.numpy as jnp
from jax.experimental import pallas as pl
from jax.experimental.pallas import tpu as pltpu


def _upsample_nearest(x, k):
    x = jnp.repeat(x, k, axis=2)
    x = jnp.repeat(x, k, axis=1)
    return x


def _conv3x3_same(x, w_taps):
    B, Hl, Wl, C = x.shape
    bf16, f32 = jnp.bfloat16, jnp.float32
    xb = x.astype(bf16)
    zr = jnp.zeros((B, 1, Wl, C), bf16)
    xp = jnp.concatenate([zr, xb, zr], axis=1)
    zc = jnp.zeros((B, Hl + 2, 1, C), bf16)
    xp = jnp.concatenate([zc, xp, zc], axis=2)
    acc = jnp.zeros((B * Hl * Wl, C), f32)
    for t, (dy, dx) in enumerate((dy, dx) for dy in range(3) for dx in range(3)):
        win = xp[:, dy:dy + Hl, dx:dx + Wl, :].reshape(B * Hl * Wl, C)
        acc = acc + jnp.dot(win, w_taps[t], preferred_element_type=f32)
    return acc.reshape(B, Hl, Wl, C)


def _avgpool2_mm(x_flat, A):
    B = x_flat.shape[0]
    xb = x_flat.astype(jnp.bfloat16)
    return jnp.stack([
        jnp.dot(A, xb[b], preferred_element_type=jnp.float32)
        for b in range(B)])


def _msfr_kernel(f_ref, w_ref, a3_ref, a4_ref,
                 p2_ref, p3_ref, p4_ref, p5_ref, p6_ref, *, B, H, W, C):
    F = f_ref[...].reshape(B, H, W, C)
    w = w_ref[...]

    p2_ref[...] = _upsample_nearest(F, 8).reshape(B, 64 * H * W, C)

    u4 = _upsample_nearest(F, 4)
    p3 = _conv3x3_same(u4, w) + u4
    p3_ref[...] = p3.reshape(B, 16 * H * W, C)

    z4 = _avgpool2_mm(p3.reshape(B, 16 * H * W, C), a3_ref[...])
    p4 = _conv3x3_same(z4.reshape(B, 2 * H, 2 * W, C), w) \
        + _upsample_nearest(F, 2)
    p4_ref[...] = p4.reshape(B, 4 * H * W, C)

    z5 = _avgpool2_mm(p4.reshape(B, 4 * H * W, C), a4_ref[...])
    p5 = _conv3x3_same(z5.reshape(B, H, W, C), w) + F
    p5_ref[...] = p5.reshape(B, H * W, C)

    Wp = W // 2
    for qy in range(H // 2):
        b0, b1 = (2 * qy) * W, (2 * qy + 1) * W
        m = jnp.maximum(
            jnp.maximum(p5_ref[:, pl.ds(b0, Wp, 2), :],
                        p5_ref[:, pl.ds(b0 + 1, Wp, 2), :]),
            jnp.maximum(p5_ref[:, pl.ds(b1, Wp, 2), :],
                        p5_ref[:, pl.ds(b1 + 1, Wp, 2), :]))
        p6_ref[:, pl.ds(qy * Wp, Wp), :] = m


def _pool_matrix(Hl, Wl):
    Pi, Po = Hl * Wl, (Hl // 2) * (Wl // 2)
    qi = jnp.arange(Po)[:, None]
    pi = jnp.arange(Pi)[None, :]
    y, x = pi // Wl, pi % Wl
    qy, qx = qi // (Wl // 2), qi % (Wl // 2)
    hit = (y // 2 == qy) & (x // 2 == qx)
    return jnp.where(hit, 0.25, 0.0).astype(jnp.bfloat16)


def _msfr_pallas(f_flat, w_taps, a3, a4, N, C, H, W, B):
    P1 = H * W

    def blk(P):
        return pl.BlockSpec((B, P, C), lambda n: (n, 0, 0))

    out_shape = (
        jax.ShapeDtypeStruct((N, 64 * P1, C), jnp.float32),
        jax.ShapeDtypeStruct((N, 16 * P1, C), jnp.float32),
        jax.ShapeDtypeStruct((N, 4 * P1, C), jnp.float32),
        jax.ShapeDtypeStruct((N, P1, C), jnp.float32),
        jax.ShapeDtypeStruct((N, P1 // 4, C), jnp.float32),
    )
    return pl.pallas_call(
        functools.partial(_msfr_kernel, B=B, H=H, W=W, C=C),
        grid=(N // B,),
        in_specs=[
            blk(P1),
            pl.BlockSpec((9, C, C), lambda n: (0, 0, 0)),
            pl.BlockSpec(a3.shape, lambda n: (0, 0)),
            pl.BlockSpec(a4.shape, lambda n: (0, 0)),
        ],
        out_specs=(blk(64 * P1), blk(16 * P1), blk(4 * P1), blk(P1),
                   blk(P1 // 4)),
        out_shape=out_shape,
        compiler_params=pltpu.CompilerParams(
            dimension_semantics=("parallel",)),
    )(f_flat, w_taps, a3, a4)


def kernel(features_nchw, w_oihw):
    N, C, H, W = features_nchw.shape
    P1 = H * W
    B = 8

    f_flat = (jnp.transpose(features_nchw, (0, 2, 3, 1))
              .reshape(N, P1, C).astype(jnp.float32))
    w_taps = (jnp.transpose(w_oihw, (2, 3, 1, 0))
              .reshape(9, C, C).astype(jnp.bfloat16))
    a3 = _pool_matrix(4 * H, 4 * W)
    a4 = _pool_matrix(2 * H, 2 * W)

    p2f, p3f, p4f, p5f, p6f = _msfr_pallas(f_flat, w_taps, a3, a4,
                                           N, C, H, W, B)

    def to_nchw(x_flat, h, w):
        return jnp.transpose(x_flat.reshape(N, h, w, C), (0, 3, 1, 2))

    return [to_nchw(p2f, 8 * H, 8 * W), to_nchw(p3f, 4 * H, 4 * W),
            to_nchw(p4f, 2 * H, 2 * W), to_nchw(p5f, H, W),
            to_nchw(p6f, H // 2, W // 2)]
